# baseline (speedup 1.0000x reference)
import functools

import jax
import jax.numpy as jnp
from jax import lax
from jax.experimental import pallas as pl
from jax.experimental.pallas import tpu as pltpu
from jax.experimental.pallas import tpu_sc as plsc

N_USERS = 50000
N_ITEMS = 50000
N = N_USERS + N_ITEMS
D = 32
H = 16
E = 1600000
N_HOPS = 3
EDGE_DROP = 0.5
MESS_DROP = 0.1

NC = 2
NT = 16
ET = E // NT
W = 400
NW = ET // W
K = 80
NK = W // K
RT = N // NT
RV = 250
NEV = RT // RV


def _sc_body(init_ref, srcg_ref, dst_ref, v3_ref, mask3_ref,
             o1_ref, o2_ref, out4_ref,
             acc, gidx0, gidx1, dstb0, dstb1, vb0, vb1, rows0, rows1,
             maskb, evb, zbuf, gsem, ssem):
    c = lax.axis_index("c")
    s = lax.axis_index("s")
    coff = c * N
    tile_rbase = s * RT

    bufs = ((gidx0, dstb0, vb0, rows0), (gidx1, dstb1, vb1, rows1))

    def zb(i, carry):
        zbuf[i, :] = jnp.zeros((H,), jnp.float32)
        return carry
    lax.fori_loop(0, RV, zb, 0)

    def init_chunk(k, carry):
        r0 = tile_rbase + k * RV
        pltpu.sync_copy(zbuf, acc.at[pl.ds(r0, RV)])
        pltpu.sync_copy(init_ref.at[pl.ds(coff + r0, RV)], evb)
        pltpu.sync_copy(evb, out4_ref.at[pl.ds(r0, RV), 0, c])
        return carry
    lax.fori_loop(0, NEV, init_chunk, 0)
    plsc.subcore_barrier()

    for hop in range(N_HOPS):
        src_tab = (init_ref, o1_ref, o2_ref)[hop]
        out_tab = (o1_ref, o2_ref, None)[hop]

        def load_idx(w, buf):
            gidxb, dstbb, vbb, _ = buf
            irow = c * (E // K) + s * (ET // K) + w * NK
            pltpu.sync_copy(srcg_ref.at[pl.ds(irow, NK)], gidxb)
            drow = s * (ET // K) + w * NK
            pltpu.sync_copy(dst_ref.at[pl.ds(drow, NK)], dstbb)
            vbase = hop * E + s * ET + w * W
            pltpu.sync_copy(v3_ref.at[pl.ds(vbase, W)], vbb)

        def fire_gather(buf):
            gidxb, _, _, rowsb = buf
            for j in range(NK):
                pltpu.async_copy(src_tab.at[gidxb.at[j]],
                                 rowsb.at[pl.ds(j * K, K)], gsem)

        def drain_gather(buf):
            gidxb, _, _, rowsb = buf
            for j in range(NK):
                pltpu.make_async_copy(src_tab.at[gidxb.at[j]],
                                      rowsb.at[pl.ds(j * K, K)],
                                      gsem).wait()

        def scale(buf):
            _, _, vbb, rowsb = buf

            @plsc.parallel_loop(0, W // 16, unroll=2)
            def _(i):
                vvec = vbb[pl.ds(i * 16, 16)]
                for e in range(16):
                    r = i * 16 + e
                    rowsb[r, :] = rowsb[r, :] * vvec[e]

        def scatter(buf):
            _, dstbb, _, rowsb = buf
            descs = [
                pltpu.async_copy(rowsb.at[pl.ds(j * K, K)],
                                 acc.at[dstbb.at[j]], ssem, add=True)
                for j in range(NK)
            ]
            for dsc in descs:
                dsc.wait()

        def window(cur, prefetch_w, other):
            drain_gather(cur)
            if prefetch_w is not None:
                load_idx(prefetch_w, other)
                fire_gather(other)
            scale(cur)
            scatter(cur)

        load_idx(0, bufs[0])
        fire_gather(bufs[0])

        def pair(t, carry):
            w0 = t * 2
            window(bufs[0], w0 + 1, bufs[1])
            window(bufs[1], w0 + 2, bufs[0])
            return carry
        lax.fori_loop(0, NW // 2 - 1, pair, 0)
        window(bufs[0], NW - 1, bufs[1])
        window(bufs[1], None, bufs[0])
        plsc.subcore_barrier()

        def evac(k, carry):
            r0 = tile_rbase + k * RV
            pltpu.sync_copy(acc.at[pl.ds(r0, RV)], evb)
            pltpu.sync_copy(zbuf, acc.at[pl.ds(r0, RV)])
            pltpu.sync_copy(
                mask3_ref.at[pl.ds(hop * NC * N + coff + r0, RV)], maskb)

            def mul(i, carry2):
                evb[i, :] = evb[i, :] * maskb[i, :]
                return carry2
            lax.fori_loop(0, RV, mul, 0)
            if out_tab is not None:
                pltpu.sync_copy(evb, out_tab.at[pl.ds(coff + r0, RV)])
            pltpu.sync_copy(evb, out4_ref.at[pl.ds(r0, RV), hop + 1, c])
            return carry
        lax.fori_loop(0, NEV, evac, 0)
        plsc.subcore_barrier()


@functools.partial(
    pl.kernel,
    out_type=[
        jax.ShapeDtypeStruct((NC * N, H), jnp.float32),
        jax.ShapeDtypeStruct((NC * N, H), jnp.float32),
        jax.ShapeDtypeStruct((N, N_HOPS + 1, NC, H), jnp.float32),
    ],
    mesh=plsc.VectorSubcoreMesh(core_axis_name="c", subcore_axis_name="s",
                                num_cores=NC, num_subcores=NT),
    scratch_types=[
        pltpu.VMEM_SHARED((N, H), jnp.float32),
        pltpu.VMEM((NK, K), jnp.int32),
        pltpu.VMEM((NK, K), jnp.int32),
        pltpu.VMEM((NK, K), jnp.int32),
        pltpu.VMEM((NK, K), jnp.int32),
        pltpu.VMEM((W,), jnp.float32),
        pltpu.VMEM((W,), jnp.float32),
        pltpu.VMEM((W, H), jnp.float32),
        pltpu.VMEM((W, H), jnp.float32),
        pltpu.VMEM((RV, H), jnp.float32),
        pltpu.VMEM((RV, H), jnp.float32),
        pltpu.VMEM((RV, H), jnp.float32),
        pltpu.SemaphoreType.DMA,
        pltpu.SemaphoreType.DMA,
    ],
    compiler_params=pltpu.CompilerParams(use_tc_tiling_on_sc=False),
)
def _graph_conv_sc(init_ref, srcg_ref, dst_ref, v3_ref, mask3_ref,
                   o1_ref, o2_ref, out4_ref,
                   acc, gidx0, gidx1, dstb0, dstb1, vb0, vb1, rows0, rows1,
                   maskb, evb, zbuf, gsem, ssem):
    _sc_body(init_ref, srcg_ref, dst_ref, v3_ref, mask3_ref,
             o1_ref, o2_ref, out4_ref,
             acc, gidx0, gidx1, dstb0, dstb1, vb0, vb1, rows0, rows1,
             maskb, evb, zbuf, gsem, ssem)


def kernel(user_embed, item_embed, edge_index, edge_values):
    all_embed = jnp.concatenate([user_embed, item_embed], axis=0)
    init_tab = jnp.concatenate([all_embed[:, :H], all_embed[:, H:]], axis=0)

    dst = edge_index[0]
    src = edge_index[1]

    base_key = jax.random.key(42)
    vs, masks = [], []
    for hop in range(N_HOPS):
        ke, km = jax.random.split(jax.random.fold_in(base_key, hop))
        u = jax.random.uniform(ke, (E,), dtype=jnp.float32)
        keep = jnp.floor(EDGE_DROP + u)
        vs.append(edge_values * keep * (1.0 / (1.0 - EDGE_DROP)))
        m = (jax.random.uniform(km, (N, D)) >= MESS_DROP).astype(jnp.float32)
        m = m * (1.0 / (1.0 - MESS_DROP))
        masks.append(jnp.concatenate([m[:, :H], m[:, H:]], axis=0))
    v3 = jnp.concatenate(vs)
    mask3 = jnp.concatenate(masks)

    srcg = jnp.concatenate([src, src + N]).reshape(NC * E // K, K)
    dst2d = dst.reshape(E // K, K)

    _, _, out4 = _graph_conv_sc(init_tab, srcg, dst2d, v3, mask3)

    embs = out4.reshape(N, N_HOPS + 1, D)
    return embs[:N_USERS], embs[N_USERS:]


# speedup vs baseline: 1.2493x; 1.2493x over previous
import functools

import jax
import jax.numpy as jnp
from jax import lax
from jax.experimental import pallas as pl
from jax.experimental.pallas import tpu as pltpu
from jax.experimental.pallas import tpu_sc as plsc

N_USERS = 50000
N_ITEMS = 50000
N = N_USERS + N_ITEMS
D = 32
H = 16
E = 1600000
N_HOPS = 3
EDGE_DROP = 0.5
MESS_DROP = 0.1

NC = 2
NT = 16
NP = 102400
EP = NT * NP
ET = EP // NT
W = 512
NW = ET // W
K = 128
NK = W // K
RT = NP // NT
RV = 128
NEV = RT // RV
MR = RV * H // 128


def _sc_body(init_ref, srcg_ref, dst_ref, v3_ref, mask3_ref,
             o1_ref, o2_ref, out4_ref,
             acc, gidx0, gidx1, dstb0, dstb1, vb0, vb1, rows0, rows1,
             maskb, evb, zbuf, gsem, isem, ssem):
    c = lax.axis_index("c")
    s = lax.axis_index("s")
    coff = c * NP
    tile_rbase = s * RT

    bufs = ((gidx0, dstb0, vb0, rows0), (gidx1, dstb1, vb1, rows1))

    def zb(i, carry):
        zbuf[i, :] = jnp.zeros((H,), jnp.float32)
        return carry
    lax.fori_loop(0, RV, zb, 0)

    def init_chunk(k, carry):
        r0 = tile_rbase + k * RV
        pltpu.sync_copy(zbuf, acc.at[pl.ds(r0, RV)])
        pltpu.sync_copy(init_ref.at[pl.ds(coff + r0, RV)], evb)
        pltpu.sync_copy(evb, out4_ref.at[pl.ds(r0, RV), pl.ds(c * H, H)])
        return carry
    lax.fori_loop(0, NEV, init_chunk, 0)
    plsc.subcore_barrier()

    for hop in range(N_HOPS):
        src_tab = (init_ref, o1_ref, o2_ref)[hop]
        out_tab = (o1_ref, o2_ref, None)[hop]

        def idx_descs(w, buf):
            gidxb, dstbb, vbb, _ = buf
            irow = c * (EP // K) + s * (ET // K) + w * NK
            drow = s * (ET // K) + w * NK
            vbase = hop * EP + s * ET + w * W
            return (
                (srcg_ref.at[pl.ds(irow, NK)], gidxb),
                (dst_ref.at[pl.ds(drow, NK)], dstbb),
                (v3_ref.at[pl.ds(vbase, W)], vbb),
            )

        def fire_idx(w, buf):
            for src, dstr in idx_descs(w, buf):
                pltpu.async_copy(src, dstr, isem)

        def drain_idx(w, buf):
            for src, dstr in idx_descs(w, buf):
                pltpu.make_async_copy(src, dstr, isem).wait()

        def gather_descs(buf):
            gidxb, _, _, rowsb = buf
            return [(src_tab.at[gidxb.at[j]], rowsb.at[pl.ds(j * K, K)])
                    for j in range(NK)]

        def fire_gather(buf):
            for src, dstr in gather_descs(buf):
                pltpu.async_copy(src, dstr, gsem)

        def drain_gather(buf):
            for src, dstr in gather_descs(buf):
                pltpu.make_async_copy(src, dstr, gsem).wait()

        def scatter_descs(buf):
            _, dstbb, _, rowsb = buf
            return [(rowsb.at[pl.ds(j * K, K)], acc.at[dstbb.at[j]])
                    for j in range(NK)]

        def fire_scatter(buf):
            return [pltpu.async_copy(src, dstr, ssem, add=True)
                    for src, dstr in scatter_descs(buf)]

        def scale(buf):
            _, _, vbb, rowsb = buf

            @plsc.parallel_loop(0, W // 16, unroll=2)
            def _(i):
                vvec = vbb[pl.ds(i * 16, 16)]
                for e in range(16):
                    r = i * 16 + e
                    rowsb[r, :] = rowsb[r, :] * vvec[e]

        def win_serial(w, carry):
            buf = bufs[0]
            for src, dstr in idx_descs(w, buf):
                pltpu.sync_copy(src, dstr)
            gdescs = [pltpu.async_copy(src, dstr, gsem)
                      for src, dstr in gather_descs(buf)]
            for dsc in gdescs:
                dsc.wait()
            scale(buf)
            sdescs = fire_scatter(buf)
            for dsc in sdescs:
                dsc.wait()
            return carry
        lax.fori_loop(0, NW, win_serial, 0)
        plsc.subcore_barrier()

        def evac(k, carry):
            r0 = tile_rbase + k * RV
            pltpu.sync_copy(acc.at[pl.ds(r0, RV)], evb)
            mrow = hop * (NC * NP * H // 128) + (coff + r0) * H // 128
            pltpu.sync_copy(mask3_ref.at[pl.ds(mrow, MR)], maskb)

            def mul(im, carry2):
                for j in range(8):
                    r = im * 8 + j
                    evb[r, :] = evb[r, :] * maskb[im, pl.ds(j * H, H)]
                return carry2
            lax.fori_loop(0, MR, mul, 0)

            pltpu.sync_copy(zbuf, acc.at[pl.ds(r0, RV)])
            if out_tab is not None:
                pltpu.sync_copy(evb, out_tab.at[pl.ds(coff + r0, RV)])
            cb = (hop + 1) * D + c * H
            pltpu.sync_copy(evb, out4_ref.at[pl.ds(r0, RV), pl.ds(cb, H)])
            return carry
        lax.fori_loop(0, NEV, evac, 0)
        plsc.subcore_barrier()


@functools.partial(
    pl.kernel,
    out_type=[
        jax.ShapeDtypeStruct((NC * NP, H), jnp.float32),
        jax.ShapeDtypeStruct((NC * NP, H), jnp.float32),
        jax.ShapeDtypeStruct((NP, (N_HOPS + 1) * D), jnp.float32),
    ],
    mesh=plsc.VectorSubcoreMesh(core_axis_name="c", subcore_axis_name="s",
                                num_cores=NC, num_subcores=NT),
    scratch_types=[
        pltpu.VMEM_SHARED((NP, H), jnp.float32),
        pltpu.VMEM((NK, K), jnp.int32),
        pltpu.VMEM((NK, K), jnp.int32),
        pltpu.VMEM((NK, K), jnp.int32),
        pltpu.VMEM((NK, K), jnp.int32),
        pltpu.VMEM((W,), jnp.float32),
        pltpu.VMEM((W,), jnp.float32),
        pltpu.VMEM((W, H), jnp.float32),
        pltpu.VMEM((W, H), jnp.float32),
        pltpu.VMEM((MR, 128), jnp.float32),
        pltpu.VMEM((RV, H), jnp.float32),
        pltpu.VMEM((RV, H), jnp.float32),
        pltpu.SemaphoreType.DMA,
        pltpu.SemaphoreType.DMA,
        pltpu.SemaphoreType.DMA,
    ],
    compiler_params=pltpu.CompilerParams(use_tc_tiling_on_sc=False),
)
def _graph_conv_sc(init_ref, srcg_ref, dst_ref, v3_ref, mask3_ref,
                   o1_ref, o2_ref, out4_ref,
                   acc, gidx0, gidx1, dstb0, dstb1, vb0, vb1, rows0, rows1,
                   maskb, evb, zbuf, gsem, isem, ssem):
    _sc_body(init_ref, srcg_ref, dst_ref, v3_ref, mask3_ref,
             o1_ref, o2_ref, out4_ref,
             acc, gidx0, gidx1, dstb0, dstb1, vb0, vb1, rows0, rows1,
             maskb, evb, zbuf, gsem, isem, ssem)


def kernel(user_embed, item_embed, edge_index, edge_values):
    all_embed = jnp.concatenate([user_embed, item_embed], axis=0)
    zpad = jnp.zeros((NP - N, H), jnp.float32)
    init_tab = jnp.concatenate(
        [all_embed[:, :H], zpad, all_embed[:, H:], zpad], axis=0)

    dst = edge_index[0]
    src = edge_index[1]
    npad = EP - E
    pad_idx = (jnp.arange(npad, dtype=jnp.int32) * 97) % N
    srcp = jnp.concatenate([src, pad_idx])
    dstp = jnp.concatenate([dst, pad_idx])

    base_key = jax.random.key(42)
    vs, masks = [], []
    vzpad = jnp.zeros((npad,), jnp.float32)
    mpad = jnp.zeros((NP - N, H), jnp.float32)
    for hop in range(N_HOPS):
        ke, km = jax.random.split(jax.random.fold_in(base_key, hop))
        u = jax.random.uniform(ke, (E,), dtype=jnp.float32)
        keep = jnp.floor(EDGE_DROP + u)
        v = edge_values * keep * (1.0 / (1.0 - EDGE_DROP))
        vs.append(jnp.concatenate([v, vzpad]))
        m = (jax.random.uniform(km, (N, D)) >= MESS_DROP).astype(jnp.float32)
        m = m * (1.0 / (1.0 - MESS_DROP))
        masks.append(jnp.concatenate([m[:, :H], mpad, m[:, H:], mpad], axis=0))
    v3 = jnp.concatenate(vs)
    mask3 = jnp.concatenate(masks).reshape(-1, 128)

    srcg = jnp.concatenate([srcp, srcp + NP]).reshape(NC * EP // K, K)
    dst2d = dstp.reshape(EP // K, K)

    _, _, out4 = _graph_conv_sc(init_tab, srcg, dst2d, v3, mask3)

    embs = out4[:N].reshape(N, N_HOPS + 1, D)
    return embs[:N_USERS], embs[N_USERS:]


# speedup vs baseline: 1.4098x; 1.1285x over previous
import functools

import jax
import jax.numpy as jnp
from jax import lax
from jax.experimental import pallas as pl
from jax.experimental.pallas import tpu as pltpu
from jax.experimental.pallas import tpu_sc as plsc

N_USERS = 50000
N_ITEMS = 50000
N = N_USERS + N_ITEMS
D = 32
H = 16
E = 1600000
N_HOPS = 3
EDGE_DROP = 0.5
MESS_DROP = 0.1

NC = 2
NT = 16
NP = 102400
EP = NT * NP
ET = EP // NT
W = 512
NW = ET // W
K = 128
NK = W // K
RT = NP // NT
RV = 128
NEV = RT // RV
MR = RV * H // 128


def _sc_body(init_ref, srcg_ref, dst_ref, v3_ref, mask3_ref,
             o1_ref, o2_ref, out4_ref,
             acc, gidx0, gidx1, dstb0, dstb1, vb0, vb1, rows0, rows1,
             maskb, evb, zbuf, gsem, isem, ssem):
    c = lax.axis_index("c")
    s = lax.axis_index("s")
    coff = c * NP
    tile_rbase = s * RT

    bufs = ((gidx0, dstb0, vb0, rows0), (gidx1, dstb1, vb1, rows1))

    def zb(i, carry):
        zbuf[i, :] = jnp.zeros((H,), jnp.float32)
        return carry
    lax.fori_loop(0, RV, zb, 0)

    def init_chunk(k, carry):
        r0 = tile_rbase + k * RV
        pltpu.sync_copy(zbuf, acc.at[pl.ds(r0, RV)])
        pltpu.sync_copy(init_ref.at[pl.ds(coff + r0, RV)], evb)
        pltpu.sync_copy(evb, out4_ref.at[pl.ds(r0, RV), pl.ds(c * H, H)])
        return carry
    lax.fori_loop(0, NEV, init_chunk, 0)
    plsc.subcore_barrier()

    for hop in range(N_HOPS):
        src_tab = (init_ref, o1_ref, o2_ref)[hop]
        out_tab = (o1_ref, o2_ref, None)[hop]

        def idx_descs(w, buf):
            gidxb, dstbb, vbb, _ = buf
            irow = c * (EP // K) + s * (ET // K) + w * NK
            drow = s * (ET // K) + w * NK
            vbase = hop * EP + s * ET + w * W
            return (
                (srcg_ref.at[pl.ds(irow, NK)], gidxb),
                (dst_ref.at[pl.ds(drow, NK)], dstbb),
                (v3_ref.at[pl.ds(vbase, W)], vbb),
            )

        def fire_idx(w, buf):
            for src, dstr in idx_descs(w, buf):
                pltpu.async_copy(src, dstr, isem)

        def drain_idx(w, buf):
            for src, dstr in idx_descs(w, buf):
                pltpu.make_async_copy(src, dstr, isem).wait()

        def gather_descs(buf):
            gidxb, _, _, rowsb = buf
            return [(src_tab.at[gidxb.at[j]], rowsb.at[pl.ds(j * K, K)])
                    for j in range(NK)]

        def fire_gather(buf):
            for src, dstr in gather_descs(buf):
                pltpu.async_copy(src, dstr, gsem)

        def drain_gather(buf):
            for src, dstr in gather_descs(buf):
                pltpu.make_async_copy(src, dstr, gsem).wait()

        def scatter_descs(buf):
            _, dstbb, _, rowsb = buf
            return [(rowsb.at[pl.ds(j * K, K)], acc.at[dstbb.at[j]])
                    for j in range(NK)]

        def fire_scatter(buf):
            return [pltpu.async_copy(src, dstr, ssem, add=True)
                    for src, dstr in scatter_descs(buf)]

        def scale(buf):
            _, _, vbb, rowsb = buf

            @plsc.parallel_loop(0, W // 16, unroll=2)
            def _(i):
                vvec = vbb[pl.ds(i * 16, 16)]
                for e in range(16):
                    r = i * 16 + e
                    rowsb[r, :] = rowsb[r, :] * vvec[e]

        def window(w, cur, other, last=False):
            drain_gather(cur)
            if not last:
                for src, dstr in idx_descs(w + 1, other):
                    pltpu.sync_copy(src, dstr)
                fire_gather(other)
            scale(cur)
            sdescs = fire_scatter(cur)
            for dsc in sdescs:
                dsc.wait()

        for src, dstr in idx_descs(0, bufs[0]):
            pltpu.sync_copy(src, dstr)
        fire_gather(bufs[0])
        window(0, bufs[0], bufs[1])

        def pair(t, carry):
            w1 = 2 * t + 1
            window(w1, bufs[1], bufs[0])
            window(w1 + 1, bufs[0], bufs[1])
            return carry
        lax.fori_loop(0, (NW - 2) // 2, pair, 0)
        window(NW - 1, bufs[1], bufs[0], last=True)
        plsc.subcore_barrier()

        def evac(k, carry):
            r0 = tile_rbase + k * RV
            pltpu.sync_copy(acc.at[pl.ds(r0, RV)], evb)
            mrow = hop * (NC * NP * H // 128) + (coff + r0) * H // 128
            pltpu.sync_copy(mask3_ref.at[pl.ds(mrow, MR)], maskb)

            def mul(im, carry2):
                for j in range(8):
                    r = im * 8 + j
                    evb[r, :] = evb[r, :] * maskb[im, pl.ds(j * H, H)]
                return carry2
            lax.fori_loop(0, MR, mul, 0)

            pltpu.sync_copy(zbuf, acc.at[pl.ds(r0, RV)])
            if out_tab is not None:
                pltpu.sync_copy(evb, out_tab.at[pl.ds(coff + r0, RV)])
            cb = (hop + 1) * D + c * H
            pltpu.sync_copy(evb, out4_ref.at[pl.ds(r0, RV), pl.ds(cb, H)])
            return carry
        lax.fori_loop(0, NEV, evac, 0)
        plsc.subcore_barrier()


@functools.partial(
    pl.kernel,
    out_type=[
        jax.ShapeDtypeStruct((NC * NP, H), jnp.float32),
        jax.ShapeDtypeStruct((NC * NP, H), jnp.float32),
        jax.ShapeDtypeStruct((NP, (N_HOPS + 1) * D), jnp.float32),
    ],
    mesh=plsc.VectorSubcoreMesh(core_axis_name="c", subcore_axis_name="s",
                                num_cores=NC, num_subcores=NT),
    scratch_types=[
        pltpu.VMEM_SHARED((NP, H), jnp.float32),
        pltpu.VMEM((NK, K), jnp.int32),
        pltpu.VMEM((NK, K), jnp.int32),
        pltpu.VMEM((NK, K), jnp.int32),
        pltpu.VMEM((NK, K), jnp.int32),
        pltpu.VMEM((W,), jnp.float32),
        pltpu.VMEM((W,), jnp.float32),
        pltpu.VMEM((W, H), jnp.float32),
        pltpu.VMEM((W, H), jnp.float32),
        pltpu.VMEM((MR, 128), jnp.float32),
        pltpu.VMEM((RV, H), jnp.float32),
        pltpu.VMEM((RV, H), jnp.float32),
        pltpu.SemaphoreType.DMA,
        pltpu.SemaphoreType.DMA,
        pltpu.SemaphoreType.DMA,
    ],
    compiler_params=pltpu.CompilerParams(use_tc_tiling_on_sc=False),
)
def _graph_conv_sc(init_ref, srcg_ref, dst_ref, v3_ref, mask3_ref,
                   o1_ref, o2_ref, out4_ref,
                   acc, gidx0, gidx1, dstb0, dstb1, vb0, vb1, rows0, rows1,
                   maskb, evb, zbuf, gsem, isem, ssem):
    _sc_body(init_ref, srcg_ref, dst_ref, v3_ref, mask3_ref,
             o1_ref, o2_ref, out4_ref,
             acc, gidx0, gidx1, dstb0, dstb1, vb0, vb1, rows0, rows1,
             maskb, evb, zbuf, gsem, isem, ssem)


def kernel(user_embed, item_embed, edge_index, edge_values):
    all_embed = jnp.concatenate([user_embed, item_embed], axis=0)
    zpad = jnp.zeros((NP - N, H), jnp.float32)
    init_tab = jnp.concatenate(
        [all_embed[:, :H], zpad, all_embed[:, H:], zpad], axis=0)

    dst = edge_index[0]
    src = edge_index[1]
    npad = EP - E
    pad_idx = (jnp.arange(npad, dtype=jnp.int32) * 97) % N
    srcp = jnp.concatenate([src, pad_idx])
    dstp = jnp.concatenate([dst, pad_idx])

    base_key = jax.random.key(42)
    vs, masks = [], []
    vzpad = jnp.zeros((npad,), jnp.float32)
    mpad = jnp.zeros((NP - N, H), jnp.float32)
    for hop in range(N_HOPS):
        ke, km = jax.random.split(jax.random.fold_in(base_key, hop))
        u = jax.random.uniform(ke, (E,), dtype=jnp.float32)
        keep = jnp.floor(EDGE_DROP + u)
        v = edge_values * keep * (1.0 / (1.0 - EDGE_DROP))
        vs.append(jnp.concatenate([v, vzpad]))
        m = (jax.random.uniform(km, (N, D)) >= MESS_DROP).astype(jnp.float32)
        m = m * (1.0 / (1.0 - MESS_DROP))
        masks.append(jnp.concatenate([m[:, :H], mpad, m[:, H:], mpad], axis=0))
    v3 = jnp.concatenate(vs)
    mask3 = jnp.concatenate(masks).reshape(-1, 128)

    srcg = jnp.concatenate([srcp, srcp + NP]).reshape(NC * EP // K, K)
    dst2d = dstp.reshape(EP // K, K)

    _, _, out4 = _graph_conv_sc(init_tab, srcg, dst2d, v3, mask3)

    embs = out4[:N].reshape(N, N_HOPS + 1, D)
    return embs[:N_USERS], embs[N_USERS:]


# speedup vs baseline: 1.9082x; 1.3535x over previous
import functools

import jax
import jax.numpy as jnp
from jax import lax
from jax.experimental import pallas as pl
from jax.experimental.pallas import tpu as pltpu
from jax.experimental.pallas import tpu_sc as plsc

N_USERS = 50000
N_ITEMS = 50000
N = N_USERS + N_ITEMS
D = 32
H = 16
E = 1600000
N_HOPS = 3
EDGE_DROP = 0.5
MESS_DROP = 0.1

NC = 2
NT = 16
NP = 102400
EP = NT * NP
ET = EP // NT
W = 512
NW = ET // W
K = 128
NK = W // K
RT = NP // NT
RV = 128
NEV = RT // RV
MR = RV * H // 128


def _sc_body(init_ref, srcg_ref, dst_ref, v3_ref, mask3_ref,
             o1_ref, o2_ref, out4_ref,
             acc, gidx0, gidx1, gidx2, gidx3, dstb0, dstb1, dstb2,
             dstb3, vb0, vb1, vb2, vb3, rows0, rows1,
             maskb, evb, zbuf, gsem, isem, ssem):
    c = lax.axis_index("c")
    s = lax.axis_index("s")
    coff = c * NP
    tile_rbase = s * RT

    ibufs = ((gidx0, dstb0, vb0), (gidx1, dstb1, vb1),
             (gidx2, dstb2, vb2), (gidx3, dstb3, vb3))
    rowsb = (rows0, rows1)

    def zb(i, carry):
        zbuf[i, :] = jnp.zeros((H,), jnp.float32)
        return carry
    lax.fori_loop(0, RV // 2, zb, 0)

    def init_chunk(k, carry):
        r0 = tile_rbase + k * RV
        pltpu.sync_copy(zbuf, acc.at[pl.ds(r0, RV // 2)])
        pltpu.sync_copy(zbuf, acc.at[pl.ds(r0 + RV // 2, RV // 2)])
        pltpu.sync_copy(init_ref.at[pl.ds(coff + r0, RV)], evb)
        pltpu.sync_copy(evb, out4_ref.at[pl.ds(r0, RV), pl.ds(c * H, H)])
        return carry
    lax.fori_loop(0, NEV, init_chunk, 0)
    plsc.subcore_barrier()

    for hop in range(N_HOPS):
        src_tab = (init_ref, o1_ref, o2_ref)[hop]
        out_tab = (o1_ref, o2_ref, None)[hop]

        def idx_descs(w, buf):
            gidxb, dstbb, vbb = buf
            irow = c * (EP // K) + s * (ET // K) + w * NK
            drow = s * (ET // K) + w * NK
            vbase = hop * EP + s * ET + w * W
            return (
                (srcg_ref.at[pl.ds(irow, NK)], gidxb),
                (dst_ref.at[pl.ds(drow, NK)], dstbb),
                (v3_ref.at[pl.ds(vbase, W)], vbb),
            )

        def fire_idx(w, buf):
            for src, dstr in idx_descs(w, buf):
                pltpu.async_copy(src, dstr, isem)

        def drain_idx(w, buf):
            for src, dstr in idx_descs(w, buf):
                pltpu.make_async_copy(src, dstr, isem).wait()

        def gather_descs(ibuf, rbuf):
            gidxb = ibuf[0]
            return [(src_tab.at[gidxb.at[j]], rbuf.at[pl.ds(j * K, K)])
                    for j in range(NK)]

        def fire_gather(ibuf, rbuf):
            for src, dstr in gather_descs(ibuf, rbuf):
                pltpu.async_copy(src, dstr, gsem)

        def drain_gather(ibuf, rbuf):
            for src, dstr in gather_descs(ibuf, rbuf):
                pltpu.make_async_copy(src, dstr, gsem).wait()

        def fire_scatter(ibuf, rbuf):
            dstbb = ibuf[1]
            return [pltpu.async_copy(rbuf.at[pl.ds(j * K, K)],
                                     acc.at[dstbb.at[j]], ssem, add=True)
                    for j in range(NK)]

        def scale(ibuf, rbuf):
            vbb = ibuf[2]

            @plsc.parallel_loop(0, W // 16, unroll=2)
            def _(i):
                vvec = vbb[pl.ds(i * 16, 16)]
                for e in range(16):
                    r = i * 16 + e
                    rowsb_r = rbuf
                    rowsb_r[r, :] = rowsb_r[r, :] * vvec[e]

        def window(w, wi, ri, fire_g=True, fire_i=True):
            cur_i = ibufs[wi]
            cur_r = rowsb[ri]
            if fire_g:
                drain_idx(w + 1, ibufs[(wi + 1) % 4])
                fire_gather(ibufs[(wi + 1) % 4], rowsb[1 - ri])
            drain_gather(cur_i, cur_r)
            scale(cur_i, cur_r)
            sdescs = fire_scatter(cur_i, cur_r)
            if fire_i:
                fire_idx(w + 2, ibufs[(wi + 2) % 4])
            for dsc in sdescs:
                dsc.wait()

        for src, dstr in idx_descs(0, ibufs[0]):
            pltpu.sync_copy(src, dstr)
        fire_gather(ibufs[0], rowsb[0])
        fire_idx(1, ibufs[1])

        def quad(t, carry):
            w0 = 4 * t
            for k in range(4):
                window(w0 + k, k, k % 2)
            return carry
        lax.fori_loop(0, NW // 4 - 1, quad, 0)
        window(NW - 4, 0, 0)
        window(NW - 3, 1, 1)
        window(NW - 2, 2, 0, fire_i=False)
        window(NW - 1, 3, 1, fire_g=False, fire_i=False)
        plsc.subcore_barrier()

        def evac(k, carry):
            r0 = tile_rbase + k * RV
            pltpu.sync_copy(acc.at[pl.ds(r0, RV)], evb)
            mrow = hop * (NC * NP * H // 128) + (coff + r0) * H // 128
            pltpu.sync_copy(mask3_ref.at[pl.ds(mrow, MR)], maskb)

            def mul(im, carry2):
                for j in range(8):
                    r = im * 8 + j
                    evb[r, :] = evb[r, :] * maskb[im, pl.ds(j * H, H)]
                return carry2
            lax.fori_loop(0, MR, mul, 0)

            pltpu.sync_copy(zbuf, acc.at[pl.ds(r0, RV // 2)])
            pltpu.sync_copy(zbuf, acc.at[pl.ds(r0 + RV // 2, RV // 2)])
            if out_tab is not None:
                pltpu.sync_copy(evb, out_tab.at[pl.ds(coff + r0, RV)])
            cb = (hop + 1) * D + c * H
            pltpu.sync_copy(evb, out4_ref.at[pl.ds(r0, RV), pl.ds(cb, H)])
            return carry
        lax.fori_loop(0, NEV, evac, 0)
        plsc.subcore_barrier()


@functools.partial(
    pl.kernel,
    out_type=[
        jax.ShapeDtypeStruct((NC * NP, H), jnp.float32),
        jax.ShapeDtypeStruct((NC * NP, H), jnp.float32),
        jax.ShapeDtypeStruct((NP, (N_HOPS + 1) * D), jnp.float32),
    ],
    mesh=plsc.VectorSubcoreMesh(core_axis_name="c", subcore_axis_name="s",
                                num_cores=NC, num_subcores=NT),
    scratch_types=[
        pltpu.VMEM_SHARED((NP, H), jnp.float32),
        pltpu.VMEM((NK, K), jnp.int32),
        pltpu.VMEM((NK, K), jnp.int32),
        pltpu.VMEM((NK, K), jnp.int32),
        pltpu.VMEM((NK, K), jnp.int32),
        pltpu.VMEM((NK, K), jnp.int32),
        pltpu.VMEM((NK, K), jnp.int32),
        pltpu.VMEM((NK, K), jnp.int32),
        pltpu.VMEM((NK, K), jnp.int32),
        pltpu.VMEM((W,), jnp.float32),
        pltpu.VMEM((W,), jnp.float32),
        pltpu.VMEM((W,), jnp.float32),
        pltpu.VMEM((W,), jnp.float32),
        pltpu.VMEM((W, H), jnp.float32),
        pltpu.VMEM((W, H), jnp.float32),
        pltpu.VMEM((MR, 128), jnp.float32),
        pltpu.VMEM((RV, H), jnp.float32),
        pltpu.VMEM((RV // 2, H), jnp.float32),
        pltpu.SemaphoreType.DMA,
        pltpu.SemaphoreType.DMA,
        pltpu.SemaphoreType.DMA,
    ],
    compiler_params=pltpu.CompilerParams(use_tc_tiling_on_sc=False),
)
def _graph_conv_sc(init_ref, srcg_ref, dst_ref, v3_ref, mask3_ref,
                   o1_ref, o2_ref, out4_ref,
                   acc, gidx0, gidx1, gidx2, gidx3, dstb0, dstb1, dstb2,
                   dstb3, vb0, vb1, vb2, vb3, rows0, rows1,
                   maskb, evb, zbuf, gsem, isem, ssem):
    _sc_body(init_ref, srcg_ref, dst_ref, v3_ref, mask3_ref,
             o1_ref, o2_ref, out4_ref,
             acc, gidx0, gidx1, gidx2, gidx3, dstb0, dstb1, dstb2,
             dstb3, vb0, vb1, vb2, vb3, rows0, rows1,
             maskb, evb, zbuf, gsem, isem, ssem)


def kernel(user_embed, item_embed, edge_index, edge_values):
    all_embed = jnp.concatenate([user_embed, item_embed], axis=0)
    zpad = jnp.zeros((NP - N, H), jnp.float32)
    init_tab = jnp.concatenate(
        [all_embed[:, :H], zpad, all_embed[:, H:], zpad], axis=0)

    dst = edge_index[0]
    src = edge_index[1]
    npad = EP - E
    pad_idx = (jnp.arange(npad, dtype=jnp.int32) * 97) % N
    srcp = jnp.concatenate([src, pad_idx])
    dstp = jnp.concatenate([dst, pad_idx])

    base_key = jax.random.key(42)
    vs, masks = [], []
    vzpad = jnp.zeros((npad,), jnp.float32)
    mpad = jnp.zeros((NP - N, H), jnp.float32)
    for hop in range(N_HOPS):
        ke, km = jax.random.split(jax.random.fold_in(base_key, hop))
        u = jax.random.uniform(ke, (E,), dtype=jnp.float32)
        keep = jnp.floor(EDGE_DROP + u)
        v = edge_values * keep * (1.0 / (1.0 - EDGE_DROP))
        vs.append(jnp.concatenate([v, vzpad]))
        m = (jax.random.uniform(km, (N, D)) >= MESS_DROP).astype(jnp.float32)
        m = m * (1.0 / (1.0 - MESS_DROP))
        masks.append(jnp.concatenate([m[:, :H], mpad, m[:, H:], mpad], axis=0))
    v3 = jnp.concatenate(vs)
    mask3 = jnp.concatenate(masks).reshape(-1, 128)

    srcg = jnp.concatenate([srcp, srcp + NP]).reshape(NC * EP // K, K)
    dst2d = dstp.reshape(EP // K, K)

    _, _, out4 = _graph_conv_sc(init_tab, srcg, dst2d, v3, mask3)

    embs = out4[:N].reshape(N, N_HOPS + 1, D)
    return embs[:N_USERS], embs[N_USERS:]
